# baseline (speedup 1.0000x reference)
import functools

import jax
import jax.numpy as jnp
from jax import lax
from jax.experimental import pallas as pl
from jax.experimental.pallas import tpu as pltpu
from jax.experimental.pallas import tpu_sc as plsc

_N = 10000
_E = 320000
_D = 128
_H4 = 4 * _D

_B = 1024
_NPAD = 10240
_NB = _NPAD // _B
_EPAD = 327680


def _sc_gather(table, idx, chunk):
    (m,) = idx.shape
    d = table.shape[1]
    info = plsc.get_sparse_core_info()
    nw = info.num_cores * info.num_subcores
    rows_pw = m // nw
    n_chunks = rows_pw // chunk
    assert rows_pw * nw == m and n_chunks * chunk == rows_pw and chunk % 8 == 0

    mesh = plsc.VectorSubcoreMesh(core_axis_name="c", subcore_axis_name="s")

    @functools.partial(
        pl.kernel,
        out_type=jax.ShapeDtypeStruct((m, d), table.dtype),
        mesh=mesh,
        scratch_types=[
            pltpu.VMEM((chunk,), jnp.int32),
            pltpu.VMEM((chunk, d), table.dtype),
            pltpu.SemaphoreType.DMA,
        ],
    )
    def k(table_hbm, idx_hbm, out_hbm, idx_v, rows_v, sem):
        wid = lax.axis_index("s") * info.num_cores + lax.axis_index("c")

        def chunk_body(j, carry):
            base = pl.multiple_of((wid * n_chunks + j) * chunk, 8)
            pltpu.sync_copy(idx_hbm.at[pl.ds(base, chunk)], idx_v)
            pltpu.async_copy(table_hbm.at[idx_v], rows_v, sem).wait()
            pltpu.sync_copy(rows_v, out_hbm.at[pl.ds(base, chunk)])
            return carry

        lax.fori_loop(0, n_chunks, chunk_body, 0)

    return k(table, idx)


def _lstm_kernel(md_ref, hin_ref, pccol_ref, pc2d_ref, wih_ref, whh_ref,
                 bias_ref, wl_ref, wr_ref, blr_ref, xs_hbm, out_ref,
                 h_sc, c_sc, buf_sc, sem, *, relu):
    b = pl.program_id(0)
    md = md_ref[0]
    base_rank = b * _B
    h_sc[...] = jnp.zeros((_B, _D), jnp.float32)
    c_sc[...] = jnp.zeros((_B, _D), jnp.float32)
    pc_col = pccol_ref[...]
    pc2d = pc2d_ref[...]
    wih = wih_ref[...]
    whh = whh_ref[...]
    bias = bias_ref[...]

    def step(t, off):
        n_t = jnp.sum((pc2d > t).astype(jnp.int32))
        start = off + jnp.minimum(base_rank, n_t)

        @pl.when(n_t > base_rank)
        def _():
            cp = pltpu.make_async_copy(xs_hbm.at[pl.ds(start, _B)], buf_sc, sem)
            cp.start()
            cp.wait()

        mask = pc_col > t
        xt = jnp.where(mask, buf_sc[...], 0.0)
        gates = (jnp.dot(xt, wih, preferred_element_type=jnp.float32)
                 + jnp.dot(h_sc[...], whh, preferred_element_type=jnp.float32)
                 + bias)
        i_g = gates[:, :_D]
        f_g = gates[:, _D:2 * _D]
        g_g = gates[:, 2 * _D:3 * _D]
        o_g = gates[:, 3 * _D:]
        c_new = (jax.nn.sigmoid(f_g) * c_sc[...]
                 + jax.nn.sigmoid(i_g) * jnp.tanh(g_g))
        h_sc[...] = jax.nn.sigmoid(o_g) * jnp.tanh(c_new)
        c_sc[...] = c_new
        return off + n_t

    lax.fori_loop(0, md, step, jnp.int32(0))

    out = (jnp.dot(h_sc[...], wl_ref[...], preferred_element_type=jnp.float32)
           + blr_ref[...]
           + jnp.dot(hin_ref[...], wr_ref[...], preferred_element_type=jnp.float32))
    if relu:
        out = jnp.maximum(out, 0.0)
    out_ref[...] = out


def _lstm_layer(h_in, xs, pccol, pc2d, md_arr, wih_t, whh_t, bias2,
                wl_t, wr_t, bl2, relu):
    grid_spec = pltpu.PrefetchScalarGridSpec(
        num_scalar_prefetch=1,
        grid=(_NB,),
        in_specs=[
            pl.BlockSpec((_B, _D), lambda b, md: (b, 0)),
            pl.BlockSpec((_B, 1), lambda b, md: (b, 0)),
            pl.BlockSpec((_NPAD // 128, 128), lambda b, md: (0, 0)),
            pl.BlockSpec((_D, _H4), lambda b, md: (0, 0)),
            pl.BlockSpec((_D, _H4), lambda b, md: (0, 0)),
            pl.BlockSpec((1, _H4), lambda b, md: (0, 0)),
            pl.BlockSpec((_D, _D), lambda b, md: (0, 0)),
            pl.BlockSpec((_D, _D), lambda b, md: (0, 0)),
            pl.BlockSpec((1, _D), lambda b, md: (0, 0)),
            pl.BlockSpec(memory_space=pl.ANY),
        ],
        out_specs=pl.BlockSpec((_B, _D), lambda b, md: (b, 0)),
        scratch_shapes=[
            pltpu.VMEM((_B, _D), jnp.float32),
            pltpu.VMEM((_B, _D), jnp.float32),
            pltpu.VMEM((_B, _D), jnp.float32),
            pltpu.SemaphoreType.DMA,
        ],
    )
    fn = pl.pallas_call(
        functools.partial(_lstm_kernel, relu=relu),
        grid_spec=grid_spec,
        out_shape=jax.ShapeDtypeStruct((_NPAD, _D), jnp.float32),
    )
    return fn(md_arr, h_in, pccol, pc2d, wih_t, whh_t, bias2, wl_t, wr_t,
              bl2, xs)


def kernel(x, edge_index,
           l1_W_ih, l1_W_hh, l1_b_ih, l1_b_hh, l1_Wl, l1_bl, l1_Wr,
           l2_W_ih, l2_W_hh, l2_b_ih, l2_b_hh, l2_Wl, l2_bl, l2_Wr,
           l3_W_ih, l3_W_hh, l3_b_ih, l3_b_hh, l3_Wl, l3_bl, l3_Wr):
    src = edge_index[0]
    dst = edge_index[1]
    order = jnp.argsort(dst)
    src_s = src[order].astype(jnp.int32)
    dst_s = dst[order]
    counts = jnp.bincount(dst_s, length=_N).astype(jnp.int32)
    starts = jnp.concatenate(
        [jnp.zeros((1,), counts.dtype), jnp.cumsum(counts)[:-1]])
    pos = (jnp.arange(_E, dtype=jnp.int32) - starts[dst_s]).astype(jnp.int32)
    max_deg = counts.max().astype(jnp.int32)

    perm = jnp.argsort(-counts).astype(jnp.int32)
    rank = jnp.zeros((_N,), jnp.int32).at[perm].set(
        jnp.arange(_N, dtype=jnp.int32))
    pcounts = counts[perm]
    n_per_step = jnp.bincount(pos, length=_E)
    off = jnp.concatenate(
        [jnp.zeros((1,), jnp.int32),
         jnp.cumsum(n_per_step)[:-1].astype(jnp.int32)])
    packed_pos = off[pos] + rank[dst_s]
    packed_src = jnp.zeros((_EPAD,), jnp.int32).at[packed_pos].set(src_s)
    gidx = rank[packed_src]
    perm_pad = jnp.concatenate([perm, jnp.zeros((_NPAD - _N,), jnp.int32)])
    rank_pad = jnp.concatenate([rank, jnp.zeros((_NPAD - _N,), jnp.int32)])
    pccol = jnp.concatenate(
        [pcounts, jnp.zeros((_NPAD - _N,), jnp.int32)]).reshape(_NPAD, 1)
    pc2d = pccol.reshape(_NPAD // 128, 128)
    md_arr = jnp.full((1,), max_deg, jnp.int32)

    h = _sc_gather(x, perm_pad, 320)
    layers = (
        (l1_W_ih, l1_W_hh, l1_b_ih, l1_b_hh, l1_Wl, l1_bl, l1_Wr, True),
        (l2_W_ih, l2_W_hh, l2_b_ih, l2_b_hh, l2_Wl, l2_bl, l2_Wr, True),
        (l3_W_ih, l3_W_hh, l3_b_ih, l3_b_hh, l3_Wl, l3_bl, l3_Wr, False),
    )
    for w_ih, w_hh, b_ih, b_hh, wl, bl, wr, relu in layers:
        xs = _sc_gather(h, gidx, 512)
        h = _lstm_layer(
            h, xs, pccol, pc2d, md_arr,
            w_ih.T, w_hh.T, (b_ih + b_hh).reshape(1, _H4),
            wl.T, wr.T, bl.reshape(1, _D), relu)
    out_pad = _sc_gather(h, rank_pad, 320)
    return out_pad[:_N]


# speedup vs baseline: 2.7135x; 2.7135x over previous
import functools

import jax
import jax.numpy as jnp
from jax import lax
from jax.experimental import pallas as pl
from jax.experimental.pallas import tpu as pltpu
from jax.experimental.pallas import tpu_sc as plsc

_N = 10000
_E = 320000
_D = 128
_H4 = 4 * _D

_B = 1024
_NPAD = 10240
_NB = _NPAD // _B
_EPAD = 327680
_OFFCAP = 16384


def _sc_gather(table, idx, chunk):
    (m,) = idx.shape
    d = table.shape[1]
    info = plsc.get_sparse_core_info()
    nw = info.num_cores * info.num_subcores
    rows_pw = m // nw
    n_chunks = rows_pw // chunk
    assert rows_pw * nw == m and n_chunks * chunk == rows_pw and chunk % 8 == 0
    assert n_chunks == 1 or n_chunks % 2 == 0

    mesh = plsc.VectorSubcoreMesh(core_axis_name="c", subcore_axis_name="s")

    @functools.partial(
        pl.kernel,
        out_type=jax.ShapeDtypeStruct((m, d), table.dtype),
        mesh=mesh,
        scratch_types=[
            pltpu.VMEM((chunk,), jnp.int32),
            pltpu.VMEM((chunk,), jnp.int32),
            pltpu.VMEM((chunk, d), table.dtype),
            pltpu.VMEM((chunk, d), table.dtype),
            pltpu.SemaphoreType.DMA((2,)),
            pltpu.SemaphoreType.DMA((2,)),
            pltpu.SemaphoreType.DMA((2,)),
        ],
    )
    def k(table_hbm, idx_hbm, out_hbm, idx0, idx1, rows0, rows1,
          sem_i, sem_g, sem_w):
        wid = lax.axis_index("s") * info.num_cores + lax.axis_index("c")
        idx_v = (idx0, idx1)
        rows_v = (rows0, rows1)

        def src_at(j):
            base = pl.multiple_of((wid * n_chunks + j) * chunk, 8)
            return pl.ds(base, chunk)

        def idx_start(j, b):
            pltpu.make_async_copy(
                idx_hbm.at[src_at(j)], idx_v[b], sem_i.at[b]).start()

        if n_chunks == 1:
            pltpu.sync_copy(idx_hbm.at[src_at(0)], idx0)
            pltpu.async_copy(table_hbm.at[idx0], rows0, sem_g.at[0]).wait()
            pltpu.sync_copy(rows0, out_hbm.at[src_at(0)])
            return

        idx_start(0, 0)
        idx_start(1, 1)

        def pair_body(g, carry):
            for b in range(2):
                j = g * 2 + b
                pltpu.make_async_copy(
                    idx_hbm.at[src_at(j)], idx_v[b], sem_i.at[b]).wait()

                @pl.when(j >= 2)
                def _():
                    pltpu.make_async_copy(
                        rows_v[b], out_hbm.at[src_at(j - 2)],
                        sem_w.at[b]).wait()

                pltpu.async_copy(
                    table_hbm.at[idx_v[b]], rows_v[b], sem_g.at[b]).wait()
                pltpu.make_async_copy(
                    rows_v[b], out_hbm.at[src_at(j)], sem_w.at[b]).start()

                @pl.when(j + 2 < n_chunks)
                def _():
                    idx_start(j + 2, b)
            return carry

        lax.fori_loop(0, n_chunks // 2, pair_body, 0)
        for b in range(2):
            j_last = n_chunks - 2 + b
            pltpu.make_async_copy(
                rows_v[b], out_hbm.at[src_at(j_last)], sem_w.at[b]).wait()

    return k(table, idx)


def _sc_prep(dst_s, src_s, pos, rank, off_cap):
    info = plsc.get_sparse_core_info()
    nw = info.num_cores * info.num_subcores
    epw = _E // nw
    nvec = epw // 16
    mesh = plsc.VectorSubcoreMesh(core_axis_name="c", subcore_axis_name="s")

    @functools.partial(
        pl.kernel,
        out_type=(jax.ShapeDtypeStruct((_E,), jnp.int32),
                  jax.ShapeDtypeStruct((_E,), jnp.int32)),
        mesh=mesh,
        compiler_params=pltpu.CompilerParams(needs_layout_passes=False),
        scratch_types=[
            pltpu.VMEM((epw,), jnp.int32),
            pltpu.VMEM((epw,), jnp.int32),
            pltpu.VMEM((epw,), jnp.int32),
            pltpu.VMEM((_N,), jnp.int32),
            pltpu.VMEM((_OFFCAP,), jnp.int32),
            pltpu.VMEM((epw,), jnp.int32),
            pltpu.VMEM((epw,), jnp.int32),
        ],
    )
    def k(dst_hbm, src_hbm, pos_hbm, rank_hbm, off_hbm, pp_hbm, rs_hbm,
          dst_v, src_v, pos_v, rank_v, off_v, pp_v, rs_v):
        wid = lax.axis_index("s") * info.num_cores + lax.axis_index("c")
        base = pl.multiple_of(wid * epw, 8)
        pltpu.sync_copy(dst_hbm.at[pl.ds(base, epw)], dst_v)
        pltpu.sync_copy(src_hbm.at[pl.ds(base, epw)], src_v)
        pltpu.sync_copy(pos_hbm.at[pl.ds(base, epw)], pos_v)
        pltpu.sync_copy(rank_hbm, rank_v)
        pltpu.sync_copy(off_hbm, off_v)

        def body(i, carry):
            sl = pl.ds(i * 16, 16)
            rd = plsc.load_gather(rank_v, [dst_v[sl]])
            rs = plsc.load_gather(rank_v, [src_v[sl]])
            op = plsc.load_gather(off_v, [pos_v[sl]])
            pp_v[sl] = op + rd
            rs_v[sl] = rs
            return carry

        lax.fori_loop(0, nvec, body, 0)
        pltpu.sync_copy(pp_v, pp_hbm.at[pl.ds(base, epw)])
        pltpu.sync_copy(rs_v, rs_hbm.at[pl.ds(base, epw)])

    return k(dst_s, src_s, pos, rank, off_cap)


def _lstm_kernel(md_ref, hin_ref, pccol_ref, pc2d_ref, w2_ref, whh_ref,
                 bias_ref, wl_ref, wr_ref, blr_ref, xs_hbm, out_ref,
                 xh_sc, buf_sc, c_sc, sem, *, relu):
    b = pl.program_id(0)
    md = md_ref[0]
    base_rank = b * _B
    c_sc[...] = jnp.zeros((_B, _D), jnp.float32)
    xh_sc[0, :, _D:] = jnp.zeros((_B, _D), jnp.bfloat16)
    pc_col = pccol_ref[...]
    pc2d = pc2d_ref[...]
    w2 = w2_ref[...]
    whh = whh_ref[...]
    bias = bias_ref[...]

    def issue(t, start):
        slot = lax.rem(t, 2)
        pltpu.make_async_copy(
            xs_hbm.at[pl.ds(start, _B)], buf_sc.at[slot],
            sem.at[slot]).start()

    n0 = jnp.sum((pc2d > 0).astype(jnp.int32))

    @pl.when(n0 > base_rank)
    def _():
        issue(0, jnp.minimum(base_rank, n0))

    def step(t, carry):
        off, n_t = carry
        off_next = off + n_t
        n_next = jnp.sum((pc2d > (t + 1)).astype(jnp.int32))
        slot = lax.rem(t, 2)

        @pl.when(jnp.logical_and(t + 1 < md, n_next > base_rank))
        def _():
            issue(t + 1, off_next + jnp.minimum(base_rank, n_next))

        def with_x():
            start = off + jnp.minimum(base_rank, n_t)
            pltpu.make_async_copy(
                xs_hbm.at[pl.ds(start, _B)], buf_sc.at[slot],
                sem.at[slot]).wait()
            xb = buf_sc[slot].astype(jnp.bfloat16)

            @pl.when(n_t >= base_rank + _B)
            def _():
                xh_sc[slot, :, :_D] = xb

            @pl.when(n_t < base_rank + _B)
            def _():
                xh_sc[slot, :, :_D] = jnp.where(
                    pc_col > t, xb, jnp.bfloat16(0))

            return jnp.dot(xh_sc[slot], w2,
                           preferred_element_type=jnp.float32) + bias

        def no_x():
            return jnp.dot(xh_sc[slot, :, _D:], whh,
                           preferred_element_type=jnp.float32) + bias

        gates = lax.cond(n_t > base_rank, with_x, no_x)
        i_g = gates[:, :_D]
        f_g = gates[:, _D:2 * _D]
        g_g = gates[:, 2 * _D:3 * _D]
        o_g = gates[:, 3 * _D:]
        c_new = (jax.nn.sigmoid(f_g) * c_sc[...]
                 + jax.nn.sigmoid(i_g) * jnp.tanh(g_g))
        h_new = jax.nn.sigmoid(o_g) * jnp.tanh(c_new)
        xh_sc[1 - slot, :, _D:] = h_new.astype(jnp.bfloat16)
        c_sc[...] = c_new
        return off_next, n_next

    lax.fori_loop(0, md, step, (jnp.int32(0), n0))

    aggr = xh_sc[lax.rem(md, 2), :, _D:]
    out = (jnp.dot(aggr, wl_ref[...], preferred_element_type=jnp.float32)
           + blr_ref[...]
           + jnp.dot(hin_ref[...], wr_ref[...],
                     preferred_element_type=jnp.float32))
    if relu:
        out = jnp.maximum(out, 0.0)
    out_ref[...] = out.astype(out_ref.dtype)


def _lstm_layer(h_in, xs, pccol, pc2d, md_arr, w2, whh_t, bias2,
                wl_t, wr_t, bl2, relu, out_dtype):
    grid_spec = pltpu.PrefetchScalarGridSpec(
        num_scalar_prefetch=1,
        grid=(_NB,),
        in_specs=[
            pl.BlockSpec((_B, _D), lambda b, md: (b, 0)),
            pl.BlockSpec((_B, 1), lambda b, md: (b, 0)),
            pl.BlockSpec((_NPAD // 128, 128), lambda b, md: (0, 0)),
            pl.BlockSpec((2 * _D, _H4), lambda b, md: (0, 0)),
            pl.BlockSpec((_D, _H4), lambda b, md: (0, 0)),
            pl.BlockSpec((1, _H4), lambda b, md: (0, 0)),
            pl.BlockSpec((_D, _D), lambda b, md: (0, 0)),
            pl.BlockSpec((_D, _D), lambda b, md: (0, 0)),
            pl.BlockSpec((1, _D), lambda b, md: (0, 0)),
            pl.BlockSpec(memory_space=pl.ANY),
        ],
        out_specs=pl.BlockSpec((_B, _D), lambda b, md: (b, 0)),
        scratch_shapes=[
            pltpu.VMEM((2, _B, 2 * _D), jnp.bfloat16),
            pltpu.VMEM((2, _B, _D), jnp.float32),
            pltpu.VMEM((_B, _D), jnp.float32),
            pltpu.SemaphoreType.DMA((2,)),
        ],
    )
    fn = pl.pallas_call(
        functools.partial(_lstm_kernel, relu=relu),
        grid_spec=grid_spec,
        out_shape=jax.ShapeDtypeStruct((_NPAD, _D), out_dtype),
    )
    return fn(md_arr, h_in, pccol, pc2d, w2, whh_t, bias2, wl_t, wr_t,
              bl2, xs)


def kernel(x, edge_index,
           l1_W_ih, l1_W_hh, l1_b_ih, l1_b_hh, l1_Wl, l1_bl, l1_Wr,
           l2_W_ih, l2_W_hh, l2_b_ih, l2_b_hh, l2_Wl, l2_bl, l2_Wr,
           l3_W_ih, l3_W_hh, l3_b_ih, l3_b_hh, l3_Wl, l3_bl, l3_Wr):
    src = edge_index[0].astype(jnp.int32)
    dst = edge_index[1].astype(jnp.int32)
    dst_s, src_s = lax.sort((dst, src), num_keys=1, is_stable=True)
    iota_e = lax.iota(jnp.int32, _E)
    seg_head = jnp.where(
        jnp.concatenate([jnp.ones((1,), jnp.bool_), dst_s[1:] != dst_s[:-1]]),
        iota_e, 0)
    pos = iota_e - lax.cummax(seg_head, axis=0)
    counts = jnp.zeros((_N,), jnp.int32).at[dst_s].add(1)
    iota_n = lax.iota(jnp.int32, _N)
    negc_s, perm = lax.sort((-counts, iota_n), num_keys=1, is_stable=True)
    pcounts = -negc_s
    rank = jnp.zeros((_N,), jnp.int32).at[perm].add(iota_n)
    max_deg = pcounts[0]

    deg_hist = jnp.zeros((_E,), jnp.int32).at[counts].add(1, mode="drop")
    n_per_step = _N - jnp.cumsum(deg_hist)
    off = jnp.concatenate(
        [jnp.zeros((1,), jnp.int32), jnp.cumsum(n_per_step)[:-1]])
    packed_pos, rank_src = lax.cond(
        max_deg <= _OFFCAP,
        lambda: _sc_prep(dst_s, src_s, pos, rank, off[:_OFFCAP]),
        lambda: (off[pos] + rank[dst_s], rank[src_s]),
    )
    gidx = jnp.zeros((_EPAD,), jnp.int32).at[packed_pos].add(rank_src)

    perm_pad = jnp.concatenate([perm, jnp.zeros((_NPAD - _N,), jnp.int32)])
    rank_pad = jnp.concatenate([rank, jnp.zeros((_NPAD - _N,), jnp.int32)])
    pccol = jnp.concatenate(
        [pcounts, jnp.zeros((_NPAD - _N,), jnp.int32)]).reshape(_NPAD, 1)
    pc2d = pccol.reshape(_NPAD // 128, 128)
    md_arr = jnp.full((1,), max_deg, jnp.int32)

    h = _sc_gather(x, perm_pad, 320)
    layers = (
        (l1_W_ih, l1_W_hh, l1_b_ih, l1_b_hh, l1_Wl, l1_bl, l1_Wr, True),
        (l2_W_ih, l2_W_hh, l2_b_ih, l2_b_hh, l2_Wl, l2_bl, l2_Wr, True),
        (l3_W_ih, l3_W_hh, l3_b_ih, l3_b_hh, l3_Wl, l3_bl, l3_Wr, False),
    )
    for w_ih, w_hh, b_ih, b_hh, wl, bl, wr, relu in layers:
        xs = _sc_gather(h, gidx, 320)
        w2 = jnp.concatenate([w_ih.T, w_hh.T], axis=0).astype(jnp.bfloat16)
        h = _lstm_layer(
            h, xs, pccol, pc2d, md_arr,
            w2, w_hh.T.astype(jnp.bfloat16),
            (b_ih + b_hh).reshape(1, _H4),
            wl.T.astype(jnp.bfloat16), wr.T,
            bl.reshape(1, _D), relu, jnp.float32)
    out_pad = _sc_gather(h, rank_pad, 320)
    return out_pad[:_N]
